# baseline (speedup 1.0000x reference)
import functools

import jax
import jax.numpy as jnp
from jax.experimental import pallas as pl
from jax.experimental.pallas import tpu as pltpu

DIM = 1024
HID = 512
E = 8
G = 4
NE = 9
SCALE = 1.0
NEG_INF = float("-inf")


def _dot_nt(a, b):
    return jax.lax.dot_general(a, b, (((1,), (1,)), ((), ())),
                               preferred_element_type=jnp.float32)


def _dot(a, b):
    return jax.lax.dot_general(a, b, (((1,), (0,)), ((), ())),
                               preferred_element_type=jnp.float32)


def _combine_weights(xb, wg, bg, bias):
    logits = _dot_nt(xb, wg) + bg
    m = jnp.max(logits, axis=1, keepdims=True)
    ex = jnp.exp(logits - m)
    scores = ex / jnp.sum(ex, axis=1, keepdims=True)
    sb = scores + bias
    col = jax.lax.broadcasted_iota(jnp.int32, sb.shape, 1)
    gcol = col // 2
    ge = jnp.zeros_like(sb)
    for g in range(G):
        sg = jnp.sum(jnp.where(gcol == g, sb, 0.0), axis=1, keepdims=True)
        ge = jnp.where(gcol == g, sg, ge)
    gmax1 = jnp.max(ge, axis=1, keepdims=True)
    g1 = jnp.min(jnp.where(ge == gmax1, gcol, G), axis=1, keepdims=True)
    ge2 = jnp.where(gcol == g1, NEG_INF, ge)
    gmax2 = jnp.max(ge2, axis=1, keepdims=True)
    g2 = jnp.min(jnp.where(ge2 == gmax2, gcol, G), axis=1, keepdims=True)
    gmask = (gcol == g1) | (gcol == g2)
    masked = jnp.where(gmask, sb, NEG_INF)
    m1 = jnp.max(masked, axis=1, keepdims=True)
    i1 = jnp.min(jnp.where(masked == m1, col, E), axis=1, keepdims=True)
    masked2 = jnp.where(col == i1, NEG_INF, masked)
    m2 = jnp.max(masked2, axis=1, keepdims=True)
    i2 = jnp.min(jnp.where(masked2 == m2, col, E), axis=1, keepdims=True)
    sel = (col == i1) | (col == i2)
    return jnp.where(sel, scores, 0.0) * SCALE


def _moe_body(wg_ref, bg_ref, bias_ref, x_ref, uw_ref, ub_ref, gw_ref,
              gb_ref, dw_ref, db_ref, out_ref, comb_ref):
    e = pl.program_id(1)
    xb = x_ref[...]

    @pl.when(e == 0)
    def _():
        comb_ref[...] = _combine_weights(xb, wg_ref[...], bg_ref[...],
                                         bias_ref[...])

    col = jax.lax.broadcasted_iota(jnp.int32, comb_ref.shape, 1)
    w_sum = jnp.sum(jnp.where(col == e, comb_ref[...], 0.0), axis=1,
                    keepdims=True)
    w = jnp.where(e < E, w_sum, 1.0)

    xb16 = xb.astype(jnp.bfloat16)
    u = _dot(xb16, uw_ref[0]) + ub_ref[0]
    g = _dot(xb16, gw_ref[0]) + gb_ref[0]
    h = u * (1.0 / (1.0 + jnp.exp(-u))) * g
    o = _dot(h.astype(jnp.bfloat16), dw_ref[0]) + db_ref[0]
    res = o * w

    @pl.when(e == 0)
    def _():
        out_ref[...] = res

    @pl.when(e > 0)
    def _():
        out_ref[...] += res


def kernel(x, Wg, bg, bias, up_w, up_b, gate_w, gate_b, down_w, down_b,
           s_up_w, s_up_b, s_gate_w, s_gate_b, s_down_w, s_down_b):
    orig_shape = x.shape
    x2 = x.reshape(-1, DIM)
    T = x2.shape[0]
    BT = 2048
    nt = T // BT

    uw = jnp.concatenate([up_w, s_up_w[None]], axis=0).astype(
        jnp.bfloat16).transpose(0, 2, 1)
    gw = jnp.concatenate([gate_w, s_gate_w[None]], axis=0).astype(
        jnp.bfloat16).transpose(0, 2, 1)
    dw = jnp.concatenate([down_w, s_down_w[None]], axis=0).astype(
        jnp.bfloat16).transpose(0, 2, 1)
    ub = jnp.concatenate([up_b, s_up_b[None]], axis=0).reshape(NE, 1, HID)
    gb = jnp.concatenate([gate_b, s_gate_b[None]], axis=0).reshape(NE, 1, HID)
    db = jnp.concatenate([down_b, s_down_b[None]], axis=0).reshape(NE, 1, DIM)
    bg2 = bg.reshape(1, E)
    bias2 = bias.reshape(1, E)

    grid = (nt, NE)
    out = pl.pallas_call(
        _moe_body,
        grid=grid,
        in_specs=[
            pl.BlockSpec((E, DIM), lambda t, e: (0, 0)),
            pl.BlockSpec((1, E), lambda t, e: (0, 0)),
            pl.BlockSpec((1, E), lambda t, e: (0, 0)),
            pl.BlockSpec((BT, DIM), lambda t, e: (t, 0)),
            pl.BlockSpec((1, DIM, HID), lambda t, e: (e, 0, 0)),
            pl.BlockSpec((1, 1, HID), lambda t, e: (e, 0, 0)),
            pl.BlockSpec((1, DIM, HID), lambda t, e: (e, 0, 0)),
            pl.BlockSpec((1, 1, HID), lambda t, e: (e, 0, 0)),
            pl.BlockSpec((1, HID, DIM), lambda t, e: (e, 0, 0)),
            pl.BlockSpec((1, 1, DIM), lambda t, e: (e, 0, 0)),
        ],
        out_specs=pl.BlockSpec((BT, DIM), lambda t, e: (t, 0)),
        out_shape=jax.ShapeDtypeStruct((T, DIM), jnp.float32),
        scratch_shapes=[pltpu.VMEM((BT, E), jnp.float32)],
        compiler_params=pltpu.CompilerParams(
            dimension_semantics=("parallel", "arbitrary")),
    )(Wg, bg2, bias2, x2, uw, ub, gw, gb, dw, db)
    return out.reshape(orig_shape)


# speedup vs baseline: 1.6379x; 1.6379x over previous
import jax
import jax.numpy as jnp
from jax.experimental import pallas as pl
from jax.experimental.pallas import tpu as pltpu

DIM = 1024
HID = 512
E = 8
G = 4
NE = 9
SCALE = 1.0
NEG_INF = float("-inf")


def _dot_nt(a, b):
    return jax.lax.dot_general(a, b, (((1,), (1,)), ((), ())),
                               preferred_element_type=jnp.float32)


def _combine_weights(xb, wg, bg, bias):
    logits = _dot_nt(xb, wg) + bg
    m = jnp.max(logits, axis=1, keepdims=True)
    ex = jnp.exp(logits - m)
    scores = ex / jnp.sum(ex, axis=1, keepdims=True)
    sb = scores + bias
    col = jax.lax.broadcasted_iota(jnp.int32, sb.shape, 1)
    gcol = col // 2
    ge = jnp.zeros_like(sb)
    for g in range(G):
        sg = jnp.sum(jnp.where(gcol == g, sb, 0.0), axis=1, keepdims=True)
        ge = jnp.where(gcol == g, sg, ge)
    gmax1 = jnp.max(ge, axis=1, keepdims=True)
    g1 = jnp.min(jnp.where(ge == gmax1, gcol, G), axis=1, keepdims=True)
    ge2 = jnp.where(gcol == g1, NEG_INF, ge)
    gmax2 = jnp.max(ge2, axis=1, keepdims=True)
    g2 = jnp.min(jnp.where(ge2 == gmax2, gcol, G), axis=1, keepdims=True)
    gmask = (gcol == g1) | (gcol == g2)
    masked = jnp.where(gmask, sb, NEG_INF)
    m1 = jnp.max(masked, axis=1, keepdims=True)
    i1 = jnp.min(jnp.where(masked == m1, col, E), axis=1, keepdims=True)
    masked2 = jnp.where(col == i1, NEG_INF, masked)
    m2 = jnp.max(masked2, axis=1, keepdims=True)
    i2 = jnp.min(jnp.where(masked2 == m2, col, E), axis=1, keepdims=True)
    sel = (col == i1) | (col == i2)
    return jnp.where(sel, scores, 0.0) * SCALE


def _swiglu(xb, uwm, ubm, gwm, gbm, dwm, dbm):
    u = _dot_nt(xb, uwm) + ubm
    g = _dot_nt(xb, gwm) + gbm
    h = u * (1.0 / (1.0 + jnp.exp(-u))) * g
    return _dot_nt(h, dwm) + dbm


def _moe_body(wg_ref, bg_ref, bias_ref, x_ref, uw_ref, ub_ref, gw_ref,
              gb_ref, dw_ref, db_ref, suw_ref, sub_ref, sgw_ref, sgb_ref,
              sdw_ref, sdb_ref, out_ref, comb_ref):
    e = pl.program_id(1)
    xb = x_ref[...]

    @pl.when(e == 0)
    def _():
        comb_ref[...] = _combine_weights(xb, wg_ref[...], bg_ref[...],
                                         bias_ref[...])

    @pl.when(e < E)
    def _():
        col = jax.lax.broadcasted_iota(jnp.int32, comb_ref.shape, 1)
        w = jnp.sum(jnp.where(col == e, comb_ref[...], 0.0), axis=1,
                    keepdims=True)
        res = _swiglu(xb, uw_ref[0], ub_ref[0], gw_ref[0], gb_ref[0],
                      dw_ref[0], db_ref[0]) * w
        out_ref[...] = jnp.where(e == 0, res, out_ref[...] + res)

    @pl.when(e == E)
    def _():
        out_ref[...] += _swiglu(xb, suw_ref[...], sub_ref[...],
                                sgw_ref[...], sgb_ref[...], sdw_ref[...],
                                sdb_ref[...])


def kernel(x, Wg, bg, bias, up_w, up_b, gate_w, gate_b, down_w, down_b,
           s_up_w, s_up_b, s_gate_w, s_gate_b, s_down_w, s_down_b):
    orig_shape = x.shape
    x2 = x.reshape(-1, DIM)
    T = x2.shape[0]
    BT = 2048
    nt = T // BT

    ub = up_b.reshape(E, 1, HID)
    gb = gate_b.reshape(E, 1, HID)
    db = down_b.reshape(E, 1, DIM)
    bg2 = bg.reshape(1, E)
    bias2 = bias.reshape(1, E)
    sub = s_up_b.reshape(1, HID)
    sgb = s_gate_b.reshape(1, HID)
    sdb = s_down_b.reshape(1, DIM)

    ecap = lambda t, e: (jnp.minimum(e, E - 1), 0, 0)
    grid = (nt, NE)
    out = pl.pallas_call(
        _moe_body,
        grid=grid,
        in_specs=[
            pl.BlockSpec((E, DIM), lambda t, e: (0, 0)),
            pl.BlockSpec((1, E), lambda t, e: (0, 0)),
            pl.BlockSpec((1, E), lambda t, e: (0, 0)),
            pl.BlockSpec((BT, DIM), lambda t, e: (t, 0)),
            pl.BlockSpec((1, HID, DIM), ecap),
            pl.BlockSpec((1, 1, HID), ecap),
            pl.BlockSpec((1, HID, DIM), ecap),
            pl.BlockSpec((1, 1, HID), ecap),
            pl.BlockSpec((1, DIM, HID), ecap),
            pl.BlockSpec((1, 1, DIM), ecap),
            pl.BlockSpec((HID, DIM), lambda t, e: (0, 0)),
            pl.BlockSpec((1, HID), lambda t, e: (0, 0)),
            pl.BlockSpec((HID, DIM), lambda t, e: (0, 0)),
            pl.BlockSpec((1, HID), lambda t, e: (0, 0)),
            pl.BlockSpec((DIM, HID), lambda t, e: (0, 0)),
            pl.BlockSpec((1, DIM), lambda t, e: (0, 0)),
        ],
        out_specs=pl.BlockSpec((BT, DIM), lambda t, e: (t, 0)),
        out_shape=jax.ShapeDtypeStruct((T, DIM), jnp.float32),
        scratch_shapes=[pltpu.VMEM((BT, E), jnp.float32)],
        compiler_params=pltpu.CompilerParams(
            dimension_semantics=("parallel", "arbitrary")),
    )(Wg, bg2, bias2, x2, up_w, ub, gate_w, gb, down_w, db,
      s_up_w, sub, s_gate_w, sgb, s_down_w, sdb)
    return out.reshape(orig_shape)
